# baseline (speedup 1.0000x reference)
import functools
import math

import jax
import jax.numpy as jnp
from jax import lax
from jax.experimental import pallas as pl
from jax.experimental.pallas import tpu as pltpu
from jax.experimental.pallas import tpu_sc as plsc

_NC = 2
_NS = 16
_NW = _NC * _NS
_LANES = 16


def _pos_encoding(seq_len, d_embed):
    pos = jnp.arange(seq_len, dtype=jnp.float32)
    denom = jnp.exp(
        -jnp.arange(0, d_embed, 2, dtype=jnp.float32) * math.log(10000.0) / d_embed
    )
    phase = pos[:, None] * denom[None, :]
    enc = jnp.zeros((seq_len, d_embed), dtype=jnp.float32)
    enc = enc.at[:, 0::2].set(jnp.sin(phase))
    enc = enc.at[:, 1::2].set(jnp.cos(phase))
    return enc


def _make_sc_embed(B, T, D, idx_cols):
    N = B * T
    rows_w = N // _NW
    seqs_mega = 8
    rows_mega = seqs_mega * T
    megas_w = rows_w // rows_mega
    idx_rows_w = rows_w // idx_cols
    idx_rows_mega = rows_mega // idx_cols
    batches_w = rows_w // T
    scale = float(math.sqrt(D))
    mesh = plsc.VectorSubcoreMesh(core_axis_name="c", subcore_axis_name="s")

    @functools.partial(
        pl.kernel,
        out_type=jax.ShapeDtypeStruct((B, T, D), jnp.float32),
        mesh=mesh,
        compiler_params=pltpu.CompilerParams(use_tc_tiling_on_sc=False),
        scratch_types=[
            pltpu.VMEM((idx_rows_w, idx_cols), jnp.int32),
            pltpu.VMEM((seqs_mega, T, D), jnp.float32),
            pltpu.VMEM((T, D), jnp.float32),
            pltpu.SemaphoreType.DMA,
        ],
    )
    def k(idx_hbm, table_hbm, pe_hbm, out_hbm, idx_v, rows_v, pe_v, sem):
        wid = lax.axis_index("s") * _NC + lax.axis_index("c")
        pltpu.sync_copy(pe_hbm, pe_v)
        pltpu.sync_copy(idx_hbm.at[pl.ds(wid * idx_rows_w, idx_rows_w)], idx_v)

        def mega_body(g, carry):
            cps = [
                pltpu.async_copy(
                    table_hbm.at[idx_v.at[g * idx_rows_mega + j]],
                    rows_v.at[j // (idx_rows_mega // seqs_mega),
                              pl.ds((j % (idx_rows_mega // seqs_mega))
                                    * idx_cols, idx_cols)],
                    sem,
                )
                for j in range(idx_rows_mega)
            ]
            for cp in cps:
                cp.wait()

            def row_body(r, carry2):
                for j in range(D // _LANES):
                    sl = pl.ds(j * _LANES, _LANES)
                    pe_j = pe_v[r, sl]
                    for s in range(seqs_mega):
                        rows_v[s, r, sl] = rows_v[s, r, sl] + pe_j
                return carry2

            lax.fori_loop(0, T, row_body, 0)
            b0 = wid * batches_w + g * seqs_mega
            pltpu.sync_copy(rows_v, out_hbm.at[pl.ds(b0, seqs_mega)])
            return carry

        lax.fori_loop(0, megas_w, mega_body, 0)

    return k


def kernel(indices, embed_weight):
    B, T = indices.shape
    V, D = embed_weight.shape
    N = B * T
    idx_cols = 100
    pe = _pos_encoding(T, D)
    table_s = embed_weight * jnp.float32(math.sqrt(D))
    idx2d = indices.reshape(N // idx_cols, idx_cols)
    return _make_sc_embed(B, T, D, idx_cols)(idx2d, table_s, pe)


# speedup vs baseline: 1.5509x; 1.5509x over previous
import functools
import math

import jax
import jax.numpy as jnp
from jax import lax
from jax.experimental import pallas as pl
from jax.experimental.pallas import tpu as pltpu
from jax.experimental.pallas import tpu_sc as plsc

_NC = 2
_NS = 16
_NW = _NC * _NS
_LANES = 16


def _pos_encoding(seq_len, d_embed):
    pos = jnp.arange(seq_len, dtype=jnp.float32)
    denom = jnp.exp(
        -jnp.arange(0, d_embed, 2, dtype=jnp.float32) * math.log(10000.0) / d_embed
    )
    phase = pos[:, None] * denom[None, :]
    enc = jnp.zeros((seq_len, d_embed), dtype=jnp.float32)
    enc = enc.at[:, 0::2].set(jnp.sin(phase))
    enc = enc.at[:, 1::2].set(jnp.cos(phase))
    return enc


def _make_sc_embed(B, T, D, idx_cols):
    N = B * T
    rows_w = N // _NW
    seqs_mega = 8
    rows_mega = seqs_mega * T
    megas_w = rows_w // rows_mega
    idx_rows_w = rows_w // idx_cols
    idx_rows_mega = rows_mega // idx_cols
    batches_w = rows_w // T
    scale = float(math.sqrt(D))
    mesh = plsc.VectorSubcoreMesh(core_axis_name="c", subcore_axis_name="s")

    @functools.partial(
        pl.kernel,
        out_type=jax.ShapeDtypeStruct((B, T, 2 * D), jnp.float32),
        mesh=mesh,
        compiler_params=pltpu.CompilerParams(use_tc_tiling_on_sc=False),
        scratch_types=[
            pltpu.VMEM((idx_rows_w, idx_cols), jnp.int32),
            pltpu.VMEM((seqs_mega, T, D), jnp.float32),
            pltpu.VMEM((T, D), jnp.float32),
            pltpu.SemaphoreType.DMA,
        ],
    )
    def k(idx_hbm, table_hbm, pe_hbm, out_hbm, idx_v, rows_v, pe_v, sem):
        wid = lax.axis_index("s") * _NC + lax.axis_index("c")
        pltpu.sync_copy(pe_hbm, pe_v)
        pltpu.sync_copy(idx_hbm.at[pl.ds(wid * idx_rows_w, idx_rows_w)], idx_v)

        def mega_body(g, carry):
            cps = [
                pltpu.async_copy(
                    table_hbm.at[idx_v.at[g * idx_rows_mega + j]],
                    rows_v.at[j // (idx_rows_mega // seqs_mega),
                              pl.ds((j % (idx_rows_mega // seqs_mega))
                                    * idx_cols, idx_cols)],
                    sem,
                )
                for j in range(idx_rows_mega)
            ]
            for cp in cps:
                cp.wait()

            def row_body(r, carry2):
                for j in range(D // _LANES):
                    sl = pl.ds(j * _LANES, _LANES)
                    pe_j = pe_v[r, sl]
                    for s in range(seqs_mega):
                        rows_v[s, r, sl] = rows_v[s, r, sl] * scale + pe_j
                return carry2

            lax.fori_loop(0, T, row_body, 0)
            b0 = wid * batches_w + g * seqs_mega
            pltpu.sync_copy(rows_v,
                            out_hbm.at[pl.ds(b0, seqs_mega), :, pl.ds(0, D)])
            return carry

        lax.fori_loop(0, megas_w, mega_body, 0)

    return k


def kernel(indices, embed_weight):
    B, T = indices.shape
    V, D = embed_weight.shape
    N = B * T
    idx_cols = 100
    pe = _pos_encoding(T, D)
    idx2d = indices.reshape(N // idx_cols, idx_cols)
    wide = _make_sc_embed(B, T, D, idx_cols)(idx2d, embed_weight, pe)
    return wide[:, :, :D]


# speedup vs baseline: 1.5729x; 1.0142x over previous
import functools
import math

import jax
import jax.numpy as jnp
from jax import lax
from jax.experimental import pallas as pl
from jax.experimental.pallas import tpu as pltpu
from jax.experimental.pallas import tpu_sc as plsc

_NC = 2
_NS = 16
_NW = _NC * _NS
_LANES = 16


def _pos_encoding(seq_len, d_embed):
    pos = jnp.arange(seq_len, dtype=jnp.float32)
    denom = jnp.exp(
        -jnp.arange(0, d_embed, 2, dtype=jnp.float32) * math.log(10000.0) / d_embed
    )
    phase = pos[:, None] * denom[None, :]
    enc = jnp.zeros((seq_len, d_embed), dtype=jnp.float32)
    enc = enc.at[:, 0::2].set(jnp.sin(phase))
    enc = enc.at[:, 1::2].set(jnp.cos(phase))
    return enc


def _make_sc_embed(B, T, D, idx_cols):
    N = B * T
    rows_w = N // _NW
    seqs_mega = 4
    rows_mega = seqs_mega * T
    megas_w = rows_w // rows_mega
    idx_rows_w = rows_w // idx_cols
    idx_rows_mega = rows_mega // idx_cols
    ipseq = idx_rows_mega // seqs_mega
    batches_w = rows_w // T
    scale = float(math.sqrt(D))
    mesh = plsc.VectorSubcoreMesh(core_axis_name="c", subcore_axis_name="s")

    @functools.partial(
        pl.kernel,
        out_type=jax.ShapeDtypeStruct((B, T, 2 * D), jnp.float32),
        mesh=mesh,
        compiler_params=pltpu.CompilerParams(use_tc_tiling_on_sc=False),
        scratch_types=[
            pltpu.VMEM((idx_rows_w, idx_cols), jnp.int32),
            pltpu.VMEM((2, seqs_mega, T, D), jnp.float32),
            pltpu.VMEM((T, D), jnp.float32),
            pltpu.SemaphoreType.DMA,
            pltpu.SemaphoreType.DMA,
            pltpu.SemaphoreType.DMA,
        ],
    )
    def k(idx_hbm, table_hbm, pe_hbm, out_hbm,
          idx_v, rows_v, pe_v, sem_g0, sem_g1, sem_o):
        wid = lax.axis_index("s") * _NC + lax.axis_index("c")
        pltpu.sync_copy(pe_hbm, pe_v)
        pltpu.sync_copy(idx_hbm.at[pl.ds(wid * idx_rows_w, idx_rows_w)], idx_v)

        def gathers(g, b):
            sem_b = sem_g0 if b == 0 else sem_g1
            return [
                pltpu.make_async_copy(
                    table_hbm.at[idx_v.at[g * idx_rows_mega + j]],
                    rows_v.at[b, j // ipseq,
                              pl.ds((j % ipseq) * idx_cols, idx_cols)],
                    sem_b,
                )
                for j in range(idx_rows_mega)
            ]

        def out_slice(g):
            b0 = wid * batches_w + g * seqs_mega
            return out_hbm.at[pl.ds(b0, seqs_mega), :, pl.ds(0, D)]

        for cp in gathers(0, 0):
            cp.start()

        def pair_body(p, carry):
            for b in range(2):
                g = 2 * p + b

                @pl.when(g >= 1)
                def _drain_prev_store():
                    pltpu.make_async_copy(
                        rows_v.at[1 - b], out_slice(g - 1), sem_o).wait()

                @pl.when(g + 1 < megas_w)
                def _fire_next_gathers():
                    for cp in gathers(g + 1, 1 - b):
                        cp.start()

                for cp in gathers(g, b):
                    cp.wait()

                def row_body(r, carry2):
                    for j in range(D // _LANES):
                        sl = pl.ds(j * _LANES, _LANES)
                        pe_j = pe_v[r, sl]
                        for s in range(seqs_mega):
                            rows_v[b, s, r, sl] = (
                                rows_v[b, s, r, sl] * scale + pe_j
                            )
                    return carry2

                lax.fori_loop(0, T, row_body, 0)
                pltpu.make_async_copy(rows_v.at[b], out_slice(g),
                                      sem_o).start()
            return carry

        lax.fori_loop(0, megas_w // 2, pair_body, 0)
        pltpu.make_async_copy(
            rows_v.at[(megas_w - 1) % 2], out_slice(megas_w - 1), sem_o
        ).wait()

    return k


def kernel(indices, embed_weight):
    B, T = indices.shape
    V, D = embed_weight.shape
    N = B * T
    idx_cols = 100
    pe = _pos_encoding(T, D)
    idx2d = indices.reshape(N // idx_cols, idx_cols)
    wide = _make_sc_embed(B, T, D, idx_cols)(idx2d, embed_weight, pe)
    return wide[:, :, :D]
